# baseline (speedup 1.0000x reference)
import functools
import math

import jax
import jax.numpy as jnp
from jax import lax
from jax.experimental import pallas as pl
from jax.experimental.pallas import tpu as pltpu
from jax.experimental.pallas import tpu_sc as plsc

_B, _C = 1024, 100000
_S = 64.0
_M2 = 0.3
_M3 = 0.2
_CM2 = math.cos(_M2)
_SM2 = math.sin(_M2)

_NC, _NS, _L = 2, 16, 16
_NW = _NC * _NS
_RPW = _B // _NW
_BM = 16


def _sc_margin_body(flat_hbm, label_hbm, v_hbm, lab_v, idx_v, x_v, out_v, sem):
    wid = lax.axis_index("s") * _NC + lax.axis_index("c")
    base = wid * _RPW
    pltpu.sync_copy(label_hbm.at[pl.ds(base, _RPW)], lab_v)
    for k in range(_RPW // _L):
        lab16 = jnp.maximum(lab_v[pl.ds(k * _L, _L)], 0)
        rows16 = (base + k * _L) + lax.iota(jnp.int32, _L)
        idx_v[pl.ds(k * _L, _L)] = rows16 * _C + lab16
    pltpu.async_copy(flat_hbm.at[idx_v], x_v, sem).wait()
    for k in range(_RPW // _L):
        x = x_v[pl.ds(k * _L, _L)]
        y = jnp.maximum(1.0 - x * x, 1e-12)
        i = lax.bitcast_convert_type(y, jnp.int32)
        r = lax.bitcast_convert_type(0x5F3759DF - (i >> 1), jnp.float32)
        for _ in range(3):
            r = r * (1.5 - 0.5 * y * r * r)
        sq = y * r
        out_v[pl.ds(k * _L, _L)] = (x * _CM2 - sq * _SM2 - _M3) * _S
    pltpu.sync_copy(out_v, v_hbm.at[pl.ds(base, _RPW)])


@functools.cache
def _sc_margin():
    return pl.kernel(
        _sc_margin_body,
        mesh=plsc.VectorSubcoreMesh(core_axis_name="c", subcore_axis_name="s"),
        out_type=jax.ShapeDtypeStruct((_B,), jnp.float32),
        scratch_types=[
            pltpu.VMEM((_RPW,), jnp.int32),
            pltpu.VMEM((_RPW,), jnp.int32),
            pltpu.VMEM((_RPW,), jnp.float32),
            pltpu.VMEM((_RPW,), jnp.float32),
            pltpu.SemaphoreType.DMA,
        ],
    )


def _tc_body(x_ref, lab_ref, v_ref, o_ref):
    i = pl.program_id(0)
    x = x_ref[...]
    labs = lab_ref[pl.ds(i * _BM, _BM), :]
    vs = v_ref[pl.ds(i * _BM, _BM), :]
    cols = lax.broadcasted_iota(jnp.int32, x.shape, 1)
    o_ref[...] = jnp.where(cols == labs, vs, x * _S)


def _tc_stream(cosine, lab2, v2):
    return pl.pallas_call(
        _tc_body,
        grid=(_B // _BM,),
        in_specs=[
            pl.BlockSpec((_BM, _C), lambda i: (i, 0)),
            pl.BlockSpec((_B, 1), lambda i: (0, 0)),
            pl.BlockSpec((_B, 1), lambda i: (0, 0)),
        ],
        out_specs=pl.BlockSpec((_BM, _C), lambda i: (i, 0)),
        out_shape=jax.ShapeDtypeStruct((_B, _C), jnp.float32),
    )(cosine, lab2, v2)


def kernel(cosine, label):
    v = _sc_margin()(cosine.reshape(_B * _C), label)
    return _tc_stream(cosine, label.reshape(_B, 1), v.reshape(_B, 1))


# speedup vs baseline: 1.0008x; 1.0008x over previous
import functools
import math

import jax
import jax.numpy as jnp
from jax import lax
from jax.experimental import pallas as pl
from jax.experimental.pallas import tpu as pltpu
from jax.experimental.pallas import tpu_sc as plsc

_B, _C = 1024, 100000
_S = 64.0
_M2 = 0.3
_M3 = 0.2
_CM2 = math.cos(_M2)
_SM2 = math.sin(_M2)

_NC, _NS, _L = 2, 16, 16
_NW = _NC * _NS
_RPW = _B // _NW
_BM = 16


def _sc_margin_body(flat_hbm, label_hbm, v_hbm, lab_v, idx_v, x_v, out_v, sem):
    wid = lax.axis_index("s") * _NC + lax.axis_index("c")
    base = wid * _RPW
    pltpu.sync_copy(label_hbm.at[pl.ds(base, _RPW)], lab_v)
    for k in range(_RPW // _L):
        lab16 = jnp.maximum(lab_v[pl.ds(k * _L, _L)], 0)
        rows16 = (base + k * _L) + lax.iota(jnp.int32, _L)
        idx_v[pl.ds(k * _L, _L)] = rows16 * _C + lab16
    pltpu.async_copy(flat_hbm.at[idx_v], x_v, sem).wait()
    for k in range(_RPW // _L):
        x = x_v[pl.ds(k * _L, _L)]
        y = jnp.maximum(1.0 - x * x, 1e-12)
        i = lax.bitcast_convert_type(y, jnp.int32)
        r = lax.bitcast_convert_type(0x5F3759DF - (i >> 1), jnp.float32)
        for _ in range(3):
            r = r * (1.5 - 0.5 * y * r * r)
        sq = y * r
        out_v[pl.ds(k * _L, _L)] = (x * _CM2 - sq * _SM2 - _M3) * _S
    pltpu.sync_copy(out_v, v_hbm.at[pl.ds(base, _RPW)])


@functools.cache
def _sc_margin():
    return pl.kernel(
        _sc_margin_body,
        mesh=plsc.VectorSubcoreMesh(core_axis_name="c", subcore_axis_name="s"),
        out_type=jax.ShapeDtypeStruct((_B,), jnp.float32),
        scratch_types=[
            pltpu.VMEM((_RPW,), jnp.int32),
            pltpu.VMEM((_RPW,), jnp.int32),
            pltpu.VMEM((_RPW,), jnp.float32),
            pltpu.VMEM((_RPW,), jnp.float32),
            pltpu.SemaphoreType.DMA,
        ],
    )


def _tc_body(lab_sref, v_sref, x_ref, o_ref):
    i = pl.program_id(0)
    o_ref[...] = x_ref[...] * _S
    for r in range(_BM):
        lab = lab_sref[i * _BM + r]

        @pl.when(lab >= 0)
        def _():
            base = pl.multiple_of((lab // 128) * 128, 128)
            lane = lab - base
            chunk = o_ref[pl.ds(r, 1), pl.ds(base, 128)]
            sel = lax.broadcasted_iota(jnp.int32, (1, 128), 1) == lane
            vv = jnp.full((1, 128), v_sref[i * _BM + r], jnp.float32)
            o_ref[pl.ds(r, 1), pl.ds(base, 128)] = jnp.where(sel, vv, chunk)


def _tc_stream(cosine, label, v):
    grid_spec = pltpu.PrefetchScalarGridSpec(
        num_scalar_prefetch=2,
        grid=(_B // _BM,),
        in_specs=[pl.BlockSpec((_BM, _C), lambda i, lab, vv: (i, 0))],
        out_specs=pl.BlockSpec((_BM, _C), lambda i, lab, vv: (i, 0)),
    )
    return pl.pallas_call(
        _tc_body,
        grid_spec=grid_spec,
        out_shape=jax.ShapeDtypeStruct((_B, _C), jnp.float32),
    )(label, v, cosine)


def kernel(cosine, label):
    v = _sc_margin()(cosine.reshape(_B * _C), label)
    return _tc_stream(cosine, label, v)


# speedup vs baseline: 4.9955x; 4.9913x over previous
import functools
import math

import jax
import jax.numpy as jnp
from jax import lax
from jax.experimental import pallas as pl
from jax.experimental.pallas import tpu as pltpu
from jax.experimental.pallas import tpu_sc as plsc

_B, _C = 1024, 100000
_S = 64.0
_M2 = 0.3
_M3 = 0.2
_CM2 = math.cos(_M2)
_SM2 = math.sin(_M2)

_NC, _NS, _L = 2, 16, 16
_NW = _NC * _NS
_H = 24
_TS = 130
_NG = _B // _L


def _margin_values(x):
    y = jnp.maximum(1.0 - x * x, 1e-12)
    i = lax.bitcast_convert_type(y, jnp.int32)
    r = lax.bitcast_convert_type(0x5F3759DF - (i >> 1), jnp.float32)
    for _ in range(3):
        r = r * (1.5 - 0.5 * y * r * r)
    sq = y * r
    return (x * _CM2 - sq * _SM2 - _M3) * _S


def _sc_body(xt_hbm, label_hbm, out_hbm, lab_v, ibuf0, ibuf1, obuf0, obuf1,
             isem0, isem1, osem0, osem1):
    wid = lax.axis_index("s") * _NC + lax.axis_index("c")
    trb = wid * 390 + jnp.minimum(wid, 20)
    rbase = trb * 8
    pltpu.sync_copy(label_hbm, lab_v)

    ibufs = (ibuf0, ibuf1)
    obufs = (obuf0, obuf1)
    isems = (isem0, isem1)
    osems = (osem0, osem1)

    def scale_and_fix(ib, ob, r0, height):
        for r in range(height):
            @plsc.parallel_loop(0, _B // _L, unroll=8)
            def _(i):
                ob[r, pl.ds(i * _L, _L)] = ib[r, pl.ds(i * _L, _L)] * _S

        big = jnp.full((_L,), 10000, jnp.int32)
        small = jnp.full((_L,), -1, jnp.int32)

        @pl.loop(0, _NG, init_carry=(big, small))
        def acc_loop(g, carry):
            amin, amax = carry
            lab16 = lab_v[pl.ds(g * _L, _L)]
            lr = lab16 - r0
            m = (lr >= 0) & (lr < height)
            amin = jnp.minimum(amin, jnp.where(m, lr, 10000))
            amax = jnp.maximum(amax, jnp.where(m, lr, -1))
            return amin, amax

        amin, amax = acc_loop
        mn = amin[0]
        mx = amax[0]
        for l in range(1, _L):
            mn = jnp.minimum(mn, amin[l])
            mx = jnp.maximum(mx, amax[l])

        @pl.when(mn <= mx)
        def _():
            @pl.loop(mn, mx + 1)
            def _(r):
                @pl.loop(0, _NG)
                def _(g):
                    lab16 = lab_v[pl.ds(g * _L, _L)]
                    hit = lab16 == (r0 + r)
                    x16 = ib[r, pl.ds(g * _L, _L)]
                    ob[r, pl.ds(g * _L, _L)] = jnp.where(
                        hit, _margin_values(x16), x16 * _S)

    def step_slot(t, j):
        r0 = rbase + t * _H
        pltpu.make_async_copy(
            xt_hbm.at[pl.ds(r0, _H), :], ibufs[j], isems[j]).wait()

        @pl.when(t >= 2)
        def _():
            pltpu.make_async_copy(
                obufs[j], out_hbm.at[pl.ds(r0, _H), :], osems[j]).wait()

        scale_and_fix(ibufs[j], obufs[j], r0, _H)

        @pl.when(t + 2 < _TS)
        def _():
            pltpu.async_copy(
                xt_hbm.at[pl.ds(r0 + 2 * _H, _H), :], ibufs[j], isems[j])

        pltpu.async_copy(obufs[j], out_hbm.at[pl.ds(r0, _H), :], osems[j])

    pltpu.async_copy(xt_hbm.at[pl.ds(rbase, _H), :], ibuf0, isem0)
    pltpu.async_copy(xt_hbm.at[pl.ds(rbase + _H, _H), :], ibuf1, isem1)

    @pl.loop(0, _TS, step=2)
    def _(t):
        step_slot(t, 0)
        step_slot(t + 1, 1)

    pltpu.make_async_copy(
        obuf0, out_hbm.at[pl.ds(rbase, _H), :], osem0).wait()
    pltpu.make_async_copy(
        obuf1, out_hbm.at[pl.ds(rbase, _H), :], osem1).wait()

    @pl.when(wid < 20)
    def _():
        r0 = rbase + _TS * _H
        pltpu.sync_copy(xt_hbm.at[pl.ds(r0, 8), :],
                        ibuf0.at[pl.ds(0, 8), :])
        scale_and_fix(ibuf0, obuf0, r0, 8)
        pltpu.sync_copy(obuf0.at[pl.ds(0, 8), :],
                        out_hbm.at[pl.ds(r0, 8), :])


@functools.cache
def _sc_combine():
    return pl.kernel(
        _sc_body,
        mesh=plsc.VectorSubcoreMesh(core_axis_name="c", subcore_axis_name="s"),
        out_type=jax.ShapeDtypeStruct((_C, _B), jnp.float32),
        scratch_types=[
            pltpu.VMEM((_B,), jnp.int32),
            pltpu.VMEM((_H, _B), jnp.float32),
            pltpu.VMEM((_H, _B), jnp.float32),
            pltpu.VMEM((_H, _B), jnp.float32),
            pltpu.VMEM((_H, _B), jnp.float32),
            pltpu.SemaphoreType.DMA,
            pltpu.SemaphoreType.DMA,
            pltpu.SemaphoreType.DMA,
            pltpu.SemaphoreType.DMA,
        ],
    )


def kernel(cosine, label):
    return _sc_combine()(cosine.T, label).T
